# baseline (speedup 1.0000x reference)
import functools

import jax
import jax.numpy as jnp
from jax import lax
from jax.experimental import pallas as pl
from jax.experimental.pallas import tpu as pltpu
from jax.experimental.pallas import tpu_sc as plsc

N = 10080
E = 322560
D = 128
DE = 16
DEXT = 144
NPG = 84
NG = 120

NC = 2
NS = 16
NW = NC * NS
EPW = E // NW
B = 112
NCH = EPW // B
EB = 80
ENCH = EPW // EB
RPS = N // NS

_mesh = plsc.VectorSubcoreMesh(core_axis_name="c", subcore_axis_name="s")


def _attr_body(ea_hbm, dst_hbm, out_hbm, outd_hbm, acc_sh, accd_sh,
               ea0, ea1, dst0, dst1, ones_v, zb_v, isem0, isem1):
    c = lax.axis_index("c")
    s = lax.axis_index("s")
    wid = s * NC + c

    ea = (ea0, ea1)
    dst = (dst0, dst1)
    isem = (isem0, isem1)

    zeros16 = jnp.zeros((16,), jnp.float32)
    ones16 = jnp.full((16,), 1.0, jnp.float32)

    def _zfill(i, _):
        ea0[i, pl.ds(0, 16)] = zeros16
        return 0
    lax.fori_loop(0, B, _zfill, 0)
    for g in range(B // 16):
        ones_v[pl.ds(g * 16, 16)] = ones16
    for g in range(6):
        zb_v[pl.ds(g * 16, 16)] = zeros16
    for k in range(5):
        pltpu.sync_copy(ea0, acc_sh.at[pl.ds(s * RPS + k * B, B)])
    pltpu.sync_copy(ea0.at[pl.ds(0, RPS - 5 * B)],
                    acc_sh.at[pl.ds(s * RPS + 5 * B, RPS - 5 * B)])
    for k in range(7):
        idx = s + 16 * k

        @pl.when(idx < N // 96)
        def _():
            pltpu.sync_copy(zb_v,
                            accd_sh.at[pl.ds(pl.multiple_of(idx * 96, 8), 96)])
    plsc.subcore_barrier()

    def _issue_idx(p, ch):
        base = wid * EPW + ch * B
        pltpu.async_copy(ea_hbm.at[pl.ds(base, B)], ea[p], isem[p])
        pltpu.async_copy(dst_hbm.at[pl.ds(base, B)], dst[p], isem[p])

    def _wait_idx(p, ch):
        base = wid * EPW + ch * B
        pltpu.make_async_copy(ea_hbm.at[pl.ds(base, B)], ea[p],
                              isem[p]).wait()
        pltpu.make_async_copy(dst_hbm.at[pl.ds(base, B)], dst[p],
                              isem[p]).wait()

    _issue_idx(0, 0)
    _issue_idx(1, 1)

    def _iter(i, _):
        for p in (0, 1):
            ch = 2 * i + p
            _wait_idx(p, ch)
            pltpu.sync_copy(ea[p], acc_sh.at[dst[p]], add=True)
            pltpu.sync_copy(ones_v, accd_sh.at[dst[p]], add=True)

            @pl.when(ch + 2 < NCH)
            def _():
                _issue_idx(p, ch + 2)
        return 0
    lax.fori_loop(0, NCH // 2, _iter, 0)

    plsc.subcore_barrier()

    @pl.when(s == 0)
    def _():
        pltpu.sync_copy(acc_sh, out_hbm.at[c])
        pltpu.sync_copy(accd_sh, outd_hbm.at[c])


_attr_scatter = functools.partial(
    pl.kernel,
    out_type=(jax.ShapeDtypeStruct((NC, N, DE), jnp.float32),
              jax.ShapeDtypeStruct((NC, N), jnp.float32)),
    mesh=_mesh,
    scratch_types=[
        pltpu.VMEM_SHARED((N, DE), jnp.float32),
        pltpu.VMEM_SHARED((N,), jnp.float32),
        pltpu.VMEM((B, DE), jnp.float32),
        pltpu.VMEM((B, DE), jnp.float32),
        pltpu.VMEM((B,), jnp.int32),
        pltpu.VMEM((B,), jnp.int32),
        pltpu.VMEM((B,), jnp.float32),
        pltpu.VMEM((96,), jnp.float32),
        pltpu.SemaphoreType.DMA,
        pltpu.SemaphoreType.DMA,
    ],
    compiler_params=pltpu.CompilerParams(use_tc_tiling_on_sc=False),
)(_attr_body)


def _edge_body(hw_hbm, hs_hbm, hd_hbm, eat_hbm, src_hbm, dst_hbm,
               out_hbm, outd_hbm,
               acc_sh, accd_sh,
               src0, src1, dst0, dst1, sdst0, sdst1, eat0, eat1,
               hsg0, hsg1, hdg0, hdg1, ex0, ex1, rows0, rows1, sca0, sca1,
               isem0, isem1, gsem0, gsem1, hsem0, hsem1, ssem0, ssem1):
    c = lax.axis_index("c")
    s = lax.axis_index("s")
    wid = s * NC + c

    src = (src0, src1)
    dst = (dst0, dst1)
    sdst = (sdst0, sdst1)
    eat = (eat0, eat1)
    hsg = (hsg0, hsg1)
    hdg = (hdg0, hdg1)
    ex = (ex0, ex1)
    rows = (rows0, rows1)
    sca = (sca0, sca1)
    isem = (isem0, isem1)
    gsem = (gsem0, gsem1)
    hsem = (hsem0, hsem1)
    ssem = (ssem0, ssem1)

    zeros16 = jnp.zeros((16,), jnp.float32)

    def _zfill(i, _):
        for j in range(8):
            sca0[i, pl.ds(j * 16, 16)] = zeros16
        return 0
    lax.fori_loop(0, EB, _zfill, 0)
    for g in range(6):
        ex0[pl.ds(g * 16, 16)] = zeros16
    for k in range(7):
        pltpu.sync_copy(sca0, acc_sh.at[pl.ds(s * RPS + k * EB, EB)])
    pltpu.sync_copy(sca0.at[pl.ds(0, RPS - 7 * EB)],
                    acc_sh.at[pl.ds(s * RPS + 7 * EB, RPS - 7 * EB)])
    for k in range(7):
        idx = s + 16 * k

        @pl.when(idx < N // 96)
        def _():
            pltpu.sync_copy(ex0.at[pl.ds(0, 96)],
                            accd_sh.at[pl.ds(pl.multiple_of(idx * 96, 8), 96)])
    plsc.subcore_barrier()

    def _issue_idx(p, ch):
        base = wid * EPW + ch * EB
        pltpu.async_copy(src_hbm.at[pl.ds(base, EB)], src[p], isem[p])
        pltpu.async_copy(dst_hbm.at[pl.ds(base, EB)], dst[p], isem[p])
        pltpu.async_copy(eat_hbm.at[pl.ds(base, EB)], eat[p], isem[p])

    def _wait_idx(p, ch):
        base = wid * EPW + ch * EB
        pltpu.make_async_copy(src_hbm.at[pl.ds(base, EB)], src[p],
                              isem[p]).wait()
        pltpu.make_async_copy(dst_hbm.at[pl.ds(base, EB)], dst[p],
                              isem[p]).wait()
        pltpu.make_async_copy(eat_hbm.at[pl.ds(base, EB)], eat[p],
                              isem[p]).wait()

    def _issue_gathers(p):
        pltpu.async_copy(hw_hbm.at[src[p]], rows[p], gsem[p])
        pltpu.async_copy(hs_hbm.at[src[p]], hsg[p], hsem[p])
        pltpu.async_copy(hd_hbm.at[dst[p]], hdg[p], hsem[p])

    def _wait_scatters(p):
        pltpu.make_async_copy(sca[p], acc_sh.at[sdst[p]], ssem[p]).wait()
        pltpu.make_async_copy(ex[p].at[pl.ds(0, EB)], accd_sh.at[sdst[p]],
                              ssem[p]).wait()

    _issue_idx(0, 0)
    _wait_idx(0, 0)
    _issue_gathers(0)
    _issue_idx(1, 1)

    def _iter(i, _):
        for p in (0, 1):
            q = 1 - p
            ch = 2 * i + p

            @pl.when(ch + 1 < ENCH)
            def _():
                _wait_idx(q, ch + 1)
                _issue_gathers(q)

            @pl.when(ch >= 2)
            def _():
                _wait_scatters(p)

            pltpu.make_async_copy(hs_hbm.at[src[p]], hsg[p], hsem[p]).wait()
            pltpu.make_async_copy(hd_hbm.at[dst[p]], hdg[p], hsem[p]).wait()
            for g in range(EB // 16):
                sl = pl.ds(g * 16, 16)
                a = hsg[p][sl] + hdg[p][sl] + eat[p][sl]
                a = jnp.where(a > 0, a, 0.2 * a)
                ex[p][sl] = jnp.exp(a)
            pltpu.make_async_copy(hw_hbm.at[src[p]], rows[p], gsem[p]).wait()
            for g in range(EB // 16):
                sl = pl.ds(g * 16, 16)
                sdst[p][sl] = dst[p][sl]

            def _edge(b, _):
                exb = ex[p][pl.ds(b, 16)][0]
                for j in range(8):
                    js = pl.ds(j * 16, 16)
                    sca[p][b, js] = rows[p][b, js] * exb
                return 0
            lax.fori_loop(0, EB, _edge, 0)

            pltpu.async_copy(sca[p], acc_sh.at[sdst[p]], ssem[p], add=True)
            pltpu.async_copy(ex[p].at[pl.ds(0, EB)], accd_sh.at[sdst[p]],
                             ssem[p], add=True)

            @pl.when(ch + 2 < ENCH)
            def _():
                _issue_idx(p, ch + 2)
        return 0
    lax.fori_loop(0, ENCH // 2, _iter, 0)

    for p in (0, 1):
        _wait_scatters(p)
    plsc.subcore_barrier()

    @pl.when(s == 0)
    def _():
        pltpu.sync_copy(acc_sh, out_hbm.at[c])
        pltpu.sync_copy(accd_sh, outd_hbm.at[c])


_edge_pass = functools.partial(
    pl.kernel,
    out_type=(jax.ShapeDtypeStruct((NC, N, D), jnp.float32),
              jax.ShapeDtypeStruct((NC, N), jnp.float32)),
    mesh=_mesh,
    scratch_types=(
        [pltpu.VMEM_SHARED((N, D), jnp.float32),
         pltpu.VMEM_SHARED((N,), jnp.float32)]
        + [pltpu.VMEM((EB,), jnp.int32)] * 6
        + [pltpu.VMEM((EB,), jnp.float32)] * 6
        + [pltpu.VMEM((EB + 16,), jnp.float32)] * 2
        + [pltpu.VMEM((EB, D), jnp.float32)] * 4
        + [pltpu.SemaphoreType.DMA] * 8
    ),
    compiler_params=pltpu.CompilerParams(use_tc_tiling_on_sc=False),
)(_edge_body)


_NB = 1008
_EB = 5120
_FB = 2016


def _hw_body(x_ref, w_ref, as_ref, ad_ref, hw_ref, hs_ref, hd_ref):
    hw = jnp.dot(x_ref[...], w_ref[...], preferred_element_type=jnp.float32)
    hw_ref[...] = hw
    hs_ref[...] = jnp.dot(hw, as_ref[...], preferred_element_type=jnp.float32)
    hd_ref[...] = jnp.dot(hw, ad_ref[...], preferred_element_type=jnp.float32)


def _k_hw(x, w, a_s, a_d):
    return pl.pallas_call(
        _hw_body,
        grid=(N // _NB,),
        in_specs=[
            pl.BlockSpec((_NB, D), lambda i: (i, 0)),
            pl.BlockSpec((D, D), lambda i: (0, 0)),
            pl.BlockSpec((D, 1), lambda i: (0, 0)),
            pl.BlockSpec((D, 1), lambda i: (0, 0)),
        ],
        out_specs=[
            pl.BlockSpec((_NB, D), lambda i: (i, 0)),
            pl.BlockSpec((_NB, 1), lambda i: (i, 0)),
            pl.BlockSpec((_NB, 1), lambda i: (i, 0)),
        ],
        out_shape=[
            jax.ShapeDtypeStruct((N, D), jnp.float32),
            jax.ShapeDtypeStruct((N, 1), jnp.float32),
            jax.ShapeDtypeStruct((N, 1), jnp.float32),
        ],
    )(x, w, a_s, a_d)


def _ve(we_all, ae_all):
    return (we_all * ae_all[:, None, :]).sum(-1)


def _eat_body(ea_ref, we_ref, ae_ref, e0_ref, e1_ref, e2_ref):
    ve = _ve(we_ref[...], ae_ref[...])
    ea = ea_ref[...]
    outs = (e0_ref, e1_ref, e2_ref)
    for l in range(3):
        outs[l][...] = lax.dot_general(ea, ve[l], (((1,), (0,)), ((), ())),
                                       preferred_element_type=jnp.float32)


def _k_eat(ea, we_all, ae_all):
    return pl.pallas_call(
        _eat_body,
        grid=(E // _EB,),
        in_specs=[
            pl.BlockSpec((_EB, DE), lambda i: (i, 0)),
            pl.BlockSpec((3, DE, D), lambda i: (0, 0, 0)),
            pl.BlockSpec((3, D), lambda i: (0, 0)),
        ],
        out_specs=[pl.BlockSpec((_EB,), lambda i: (i,))] * 3,
        out_shape=[jax.ShapeDtypeStruct((E,), jnp.float32)] * 3,
    )(ea, we_all, ae_all)


def _split_body(ei_ref, src_ref, dst_ref):
    src_ref[...] = ei_ref[0]
    dst_ref[...] = ei_ref[1]


def _k_split(ei):
    return pl.pallas_call(
        _split_body,
        grid=(E // _EB,),
        in_specs=[pl.BlockSpec((2, _EB), lambda i: (0, i))],
        out_specs=[pl.BlockSpec((_EB,), lambda i: (i,))] * 2,
        out_shape=[jax.ShapeDtypeStruct((E,), jnp.int32)] * 2,
    )(ei)


def _lat_body(a_ref, d_ref, we_ref, ae_ref, lat_ref):
    ve = _ve(we_ref[...], ae_ref[...])
    a = a_ref[0] + a_ref[1]
    deg = d_ref[0] + d_ref[1]
    la = a / jnp.maximum(deg, 1.0)
    lat_ref[...] = lax.dot_general(la, ve, (((1,), (1,)), ((), ())),
                                   preferred_element_type=jnp.float32)


def _k_lat(a_parts, d_parts, we_all, ae_all):
    return pl.pallas_call(
        _lat_body,
        grid=(N // _NB,),
        in_specs=[
            pl.BlockSpec((NC, _NB, DE), lambda i: (0, i, 0)),
            pl.BlockSpec((NC, _NB, 1), lambda i: (0, i, 0)),
            pl.BlockSpec((3, DE, D), lambda i: (0, 0, 0)),
            pl.BlockSpec((3, D), lambda i: (0, 0)),
        ],
        out_specs=pl.BlockSpec((_NB, 3), lambda i: (i, 0)),
        out_shape=jax.ShapeDtypeStruct((N, 3), jnp.float32),
    )(a_parts, d_parts, we_all, ae_all)


def _node_out(s_ref, d_ref, hw_ref, hs_ref, hd_ref, lat_ref, b_ref, layer):
    exs = hs_ref[...] + hd_ref[...] + lat_ref[:, layer:layer + 1]
    exs = jnp.exp(jnp.where(exs > 0, exs, 0.2 * exs))
    den = d_ref[0] + d_ref[1] + exs
    num = s_ref[0] + s_ref[1] + exs * hw_ref[...]
    return num / (den + 1e-16) + b_ref[...]


def _combine_body(s_ref, d_ref, hw_ref, hs_ref, hd_ref, lat_ref, b_ref, w_ref,
                  as_ref, ad_ref, hw2_ref, hs2_ref, hd2_ref, *, layer):
    h = jnp.maximum(_node_out(s_ref, d_ref, hw_ref, hs_ref, hd_ref, lat_ref,
                              b_ref, layer), 0.0)
    hw2 = jnp.dot(h, w_ref[...], preferred_element_type=jnp.float32)
    hw2_ref[...] = hw2
    hs2_ref[...] = jnp.dot(hw2, as_ref[...], preferred_element_type=jnp.float32)
    hd2_ref[...] = jnp.dot(hw2, ad_ref[...], preferred_element_type=jnp.float32)


def _k_combine(layer, s_parts, d_parts, hw, hs, hd, lat, b, w_next, as_next,
               ad_next):
    return pl.pallas_call(
        functools.partial(_combine_body, layer=layer),
        grid=(N // _NB,),
        in_specs=[
            pl.BlockSpec((NC, _NB, D), lambda i: (0, i, 0)),
            pl.BlockSpec((NC, _NB, 1), lambda i: (0, i, 0)),
            pl.BlockSpec((_NB, D), lambda i: (i, 0)),
            pl.BlockSpec((_NB, 1), lambda i: (i, 0)),
            pl.BlockSpec((_NB, 1), lambda i: (i, 0)),
            pl.BlockSpec((_NB, 3), lambda i: (i, 0)),
            pl.BlockSpec((1, D), lambda i: (0, 0)),
            pl.BlockSpec((D, D), lambda i: (0, 0)),
            pl.BlockSpec((D, 1), lambda i: (0, 0)),
            pl.BlockSpec((D, 1), lambda i: (0, 0)),
        ],
        out_specs=[
            pl.BlockSpec((_NB, D), lambda i: (i, 0)),
            pl.BlockSpec((_NB, 1), lambda i: (i, 0)),
            pl.BlockSpec((_NB, 1), lambda i: (i, 0)),
        ],
        out_shape=[
            jax.ShapeDtypeStruct((N, D), jnp.float32),
            jax.ShapeDtypeStruct((N, 1), jnp.float32),
            jax.ShapeDtypeStruct((N, 1), jnp.float32),
        ],
    )(s_parts, d_parts, hw, hs, hd, lat, b, w_next, as_next, ad_next)


def _final_body(s_ref, d_ref, hw_ref, hs_ref, hd_ref, lat_ref, b_ref, wl_ref,
                bl_ref, out_ref):
    h = _node_out(s_ref, d_ref, hw_ref, hs_ref, hd_ref, lat_ref, b_ref, 2)
    pooled = h.reshape(_FB // NPG, NPG, D).sum(axis=1)
    out_ref[...] = jnp.maximum(
        jnp.dot(pooled, wl_ref[...], preferred_element_type=jnp.float32)
        + bl_ref[...], 0.0)


def _k_final(s_parts, d_parts, hw, hs, hd, lat, b, wl, bl):
    return pl.pallas_call(
        _final_body,
        grid=(N // _FB,),
        in_specs=[
            pl.BlockSpec((NC, _FB, D), lambda i: (0, i, 0)),
            pl.BlockSpec((NC, _FB, 1), lambda i: (0, i, 0)),
            pl.BlockSpec((_FB, D), lambda i: (i, 0)),
            pl.BlockSpec((_FB, 1), lambda i: (i, 0)),
            pl.BlockSpec((_FB, 1), lambda i: (i, 0)),
            pl.BlockSpec((_FB, 3), lambda i: (i, 0)),
            pl.BlockSpec((1, D), lambda i: (0, 0)),
            pl.BlockSpec((D, 1), lambda i: (0, 0)),
            pl.BlockSpec((1, 1), lambda i: (0, 0)),
        ],
        out_specs=pl.BlockSpec((_FB // NPG, 1), lambda i: (i, 0)),
        out_shape=jax.ShapeDtypeStruct((NG, 1), jnp.float32),
    )(s_parts, d_parts, hw, hs, hd, lat, b, wl, bl)


def kernel(x, edge_index, edge_attr, W0, as0, ad0, We0, ae0, b0,
           W1, as1, ad1, We1, ae1, b1, W2, as2, ad2, We2, ae2, b2, Wl, bl):
    src, dst = _k_split(edge_index)
    we_all = jnp.stack([We0, We1, We2])
    ae_all = jnp.stack([ae0, ae1, ae2])

    a_parts, ad_parts = _attr_scatter(edge_attr, dst)
    lat = _k_lat(a_parts, ad_parts.reshape(NC, N, 1), we_all, ae_all)
    eat = _k_eat(edge_attr, we_all, ae_all)

    hw, hs, hd = _k_hw(x, W0, as0.reshape(D, 1), ad0.reshape(D, 1))

    ws = [W1, W2]
    ass = [as1, as2]
    ads = [ad1, ad2]
    bs = [b0, b1, b2]
    for l in range(2):
        s_parts, d_parts = _edge_pass(hw, hs.reshape(N), hd.reshape(N),
                                      eat[l], src, dst)
        hw, hs, hd = _k_combine(l, s_parts, d_parts.reshape(NC, N, 1), hw,
                                hs, hd, lat, bs[l].reshape(1, D), ws[l],
                                ass[l].reshape(D, 1), ads[l].reshape(D, 1))
    s_parts, d_parts = _edge_pass(hw, hs.reshape(N), hd.reshape(N), eat[2],
                                  src, dst)
    return _k_final(s_parts, d_parts.reshape(NC, N, 1), hw, hs, hd, lat,
                    b2.reshape(1, D), Wl, bl.reshape(1, 1))


# speedup vs baseline: 1.5914x; 1.5914x over previous
import functools

import jax
import jax.numpy as jnp
from jax import lax
from jax.experimental import pallas as pl
from jax.experimental.pallas import tpu as pltpu
from jax.experimental.pallas import tpu_sc as plsc

N = 10080
E = 322560
D = 128
DE = 16
DEXT = 144
NPG = 84
NG = 120

NC = 2
NS = 16
NW = NC * NS
EPW = E // NW
B = 112
NCH = EPW // B
EB = 80
ENCH = EPW // EB
RPS = N // NS

_mesh = plsc.VectorSubcoreMesh(core_axis_name="c", subcore_axis_name="s")


def _attr_body(e0_hbm, e1_hbm, e2_hbm, dst_hbm, out_hbm,
               acc0, acc1, acc2, accd,
               dst0, dst1, f00, f01, f10, f11, f20, f21, ones_v, zb_v,
               isem0, isem1):
    c = lax.axis_index("c")
    s = lax.axis_index("s")
    wid = s * NC + c

    dst = (dst0, dst1)
    fs = ((f00, f01), (f10, f11), (f20, f21))
    e_hbm = (e0_hbm, e1_hbm, e2_hbm)
    accs = (acc0, acc1, acc2, accd)
    isem = (isem0, isem1)

    zeros16 = jnp.zeros((16,), jnp.float32)
    ones16 = jnp.full((16,), 1.0, jnp.float32)

    for g in range(B // 16):
        ones_v[pl.ds(g * 16, 16)] = ones16
    for g in range(6):
        zb_v[pl.ds(g * 16, 16)] = zeros16
    for k in range(7):
        idx = s + 16 * k

        @pl.when(idx < N // 96)
        def _():
            off = pl.multiple_of(idx * 96, 8)
            for a in accs:
                pltpu.sync_copy(zb_v, a.at[pl.ds(off, 96)])
    plsc.subcore_barrier()

    def _issue_idx(p, ch):
        base = wid * EPW + ch * B
        pltpu.async_copy(dst_hbm.at[pl.ds(base, B)], dst[p], isem[p])
        for l in range(3):
            pltpu.async_copy(e_hbm[l].at[pl.ds(base, B)], fs[l][p], isem[p])

    def _wait_idx(p, ch):
        base = wid * EPW + ch * B
        pltpu.make_async_copy(dst_hbm.at[pl.ds(base, B)], dst[p],
                              isem[p]).wait()
        for l in range(3):
            pltpu.make_async_copy(e_hbm[l].at[pl.ds(base, B)], fs[l][p],
                                  isem[p]).wait()

    _issue_idx(0, 0)
    _issue_idx(1, 1)

    def _iter(i, _):
        for p in (0, 1):
            ch = 2 * i + p
            _wait_idx(p, ch)
            for l in range(3):
                pltpu.sync_copy(fs[l][p], accs[l].at[dst[p]], add=True)
            pltpu.sync_copy(ones_v, accd.at[dst[p]], add=True)

            @pl.when(ch + 2 < NCH)
            def _():
                _issue_idx(p, ch + 2)
        return 0
    lax.fori_loop(0, NCH // 2, _iter, 0)

    plsc.subcore_barrier()

    @pl.when(s == 0)
    def _():
        for l in range(4):
            pltpu.sync_copy(accs[l], out_hbm.at[c, l])


_attr_scatter = functools.partial(
    pl.kernel,
    out_type=jax.ShapeDtypeStruct((NC, 4, N), jnp.float32),
    mesh=_mesh,
    scratch_types=(
        [pltpu.VMEM_SHARED((N,), jnp.float32)] * 4
        + [pltpu.VMEM((B,), jnp.int32)] * 2
        + [pltpu.VMEM((B,), jnp.float32)] * 7
        + [pltpu.VMEM((96,), jnp.float32)]
        + [pltpu.SemaphoreType.DMA] * 2
    ),
    compiler_params=pltpu.CompilerParams(use_tc_tiling_on_sc=False),
)(_attr_body)


def _edge_body(hw_hbm, hs_hbm, hd_hbm, eat_hbm, src_hbm, dst_hbm,
               out_hbm, outd_hbm,
               acc_sh, accd_sh,
               src0, src1, dst0, dst1, sdst0, sdst1, eat0, eat1,
               hsg0, hsg1, hdg0, hdg1, ex0, ex1, rows0, rows1, sca0, sca1,
               isem0, isem1, gsem0, gsem1, hsem0, hsem1, ssem0, ssem1):
    c = lax.axis_index("c")
    s = lax.axis_index("s")
    wid = s * NC + c

    src = (src0, src1)
    dst = (dst0, dst1)
    sdst = (sdst0, sdst1)
    eat = (eat0, eat1)
    hsg = (hsg0, hsg1)
    hdg = (hdg0, hdg1)
    ex = (ex0, ex1)
    rows = (rows0, rows1)
    sca = (sca0, sca1)
    isem = (isem0, isem1)
    gsem = (gsem0, gsem1)
    hsem = (hsem0, hsem1)
    ssem = (ssem0, ssem1)

    zeros16 = jnp.zeros((16,), jnp.float32)

    def _zfill(i, _):
        for j in range(8):
            sca0[i, pl.ds(j * 16, 16)] = zeros16
        return 0
    lax.fori_loop(0, EB, _zfill, 0)
    for g in range(6):
        ex0[pl.ds(g * 16, 16)] = zeros16
    for k in range(7):
        pltpu.sync_copy(sca0, acc_sh.at[pl.ds(s * RPS + k * EB, EB)])
    pltpu.sync_copy(sca0.at[pl.ds(0, RPS - 7 * EB)],
                    acc_sh.at[pl.ds(s * RPS + 7 * EB, RPS - 7 * EB)])
    for k in range(7):
        idx = s + 16 * k

        @pl.when(idx < N // 96)
        def _():
            pltpu.sync_copy(ex0.at[pl.ds(0, 96)],
                            accd_sh.at[pl.ds(pl.multiple_of(idx * 96, 8), 96)])
    plsc.subcore_barrier()

    def _issue_idx(p, ch):
        base = wid * EPW + ch * EB
        pltpu.async_copy(src_hbm.at[pl.ds(base, EB)], src[p], isem[p])
        pltpu.async_copy(dst_hbm.at[pl.ds(base, EB)], dst[p], isem[p])
        pltpu.async_copy(eat_hbm.at[pl.ds(base, EB)], eat[p], isem[p])

    def _wait_idx(p, ch):
        base = wid * EPW + ch * EB
        pltpu.make_async_copy(src_hbm.at[pl.ds(base, EB)], src[p],
                              isem[p]).wait()
        pltpu.make_async_copy(dst_hbm.at[pl.ds(base, EB)], dst[p],
                              isem[p]).wait()
        pltpu.make_async_copy(eat_hbm.at[pl.ds(base, EB)], eat[p],
                              isem[p]).wait()

    def _issue_gathers(p):
        pltpu.async_copy(hw_hbm.at[src[p]], rows[p], gsem[p])
        pltpu.async_copy(hs_hbm.at[src[p]], hsg[p], hsem[p])
        pltpu.async_copy(hd_hbm.at[dst[p]], hdg[p], hsem[p])

    def _wait_scatters(p):
        pltpu.make_async_copy(sca[p], acc_sh.at[sdst[p]], ssem[p]).wait()
        pltpu.make_async_copy(ex[p].at[pl.ds(0, EB)], accd_sh.at[sdst[p]],
                              ssem[p]).wait()

    _issue_idx(0, 0)
    _wait_idx(0, 0)
    _issue_gathers(0)
    _issue_idx(1, 1)

    def _iter(i, _):
        for p in (0, 1):
            q = 1 - p
            ch = 2 * i + p

            @pl.when(ch + 1 < ENCH)
            def _():
                _wait_idx(q, ch + 1)
                _issue_gathers(q)

            @pl.when(ch >= 2)
            def _():
                _wait_scatters(p)

            pltpu.make_async_copy(hs_hbm.at[src[p]], hsg[p], hsem[p]).wait()
            pltpu.make_async_copy(hd_hbm.at[dst[p]], hdg[p], hsem[p]).wait()
            for g in range(EB // 16):
                sl = pl.ds(g * 16, 16)
                a = hsg[p][sl] + hdg[p][sl] + eat[p][sl]
                a = jnp.where(a > 0, a, 0.2 * a)
                ex[p][sl] = jnp.exp(a)
            pltpu.make_async_copy(hw_hbm.at[src[p]], rows[p], gsem[p]).wait()
            for g in range(EB // 16):
                sl = pl.ds(g * 16, 16)
                sdst[p][sl] = dst[p][sl]

            def _edge(b, _):
                exb = ex[p][pl.ds(b, 16)][0]
                for j in range(8):
                    js = pl.ds(j * 16, 16)
                    sca[p][b, js] = rows[p][b, js] * exb
                return 0
            lax.fori_loop(0, EB, _edge, 0)

            pltpu.async_copy(sca[p], acc_sh.at[sdst[p]], ssem[p], add=True)
            pltpu.async_copy(ex[p].at[pl.ds(0, EB)], accd_sh.at[sdst[p]],
                             ssem[p], add=True)

            @pl.when(ch + 2 < ENCH)
            def _():
                _issue_idx(p, ch + 2)
        return 0
    lax.fori_loop(0, ENCH // 2, _iter, 0)

    for p in (0, 1):
        _wait_scatters(p)
    plsc.subcore_barrier()

    @pl.when(s == 0)
    def _():
        pltpu.sync_copy(acc_sh, out_hbm.at[c])
        pltpu.sync_copy(accd_sh, outd_hbm.at[c])


_edge_pass = functools.partial(
    pl.kernel,
    out_type=(jax.ShapeDtypeStruct((NC, N, D), jnp.float32),
              jax.ShapeDtypeStruct((NC, N), jnp.float32)),
    mesh=_mesh,
    scratch_types=(
        [pltpu.VMEM_SHARED((N, D), jnp.float32),
         pltpu.VMEM_SHARED((N,), jnp.float32)]
        + [pltpu.VMEM((EB,), jnp.int32)] * 6
        + [pltpu.VMEM((EB,), jnp.float32)] * 6
        + [pltpu.VMEM((EB + 16,), jnp.float32)] * 2
        + [pltpu.VMEM((EB, D), jnp.float32)] * 4
        + [pltpu.SemaphoreType.DMA] * 8
    ),
    compiler_params=pltpu.CompilerParams(use_tc_tiling_on_sc=False),
)(_edge_body)


_NB = 1008
_EB = 5120
_FB = 2016


def _hw_body(x_ref, w_ref, as_ref, ad_ref, hw_ref, hs_ref, hd_ref):
    hw = jnp.dot(x_ref[...], w_ref[...], preferred_element_type=jnp.float32)
    hw_ref[...] = hw
    hs_ref[...] = jnp.dot(hw, as_ref[...], preferred_element_type=jnp.float32)
    hd_ref[...] = jnp.dot(hw, ad_ref[...], preferred_element_type=jnp.float32)


def _k_hw(x, w, a_s, a_d):
    return pl.pallas_call(
        _hw_body,
        grid=(N // _NB,),
        in_specs=[
            pl.BlockSpec((_NB, D), lambda i: (i, 0)),
            pl.BlockSpec((D, D), lambda i: (0, 0)),
            pl.BlockSpec((D, 1), lambda i: (0, 0)),
            pl.BlockSpec((D, 1), lambda i: (0, 0)),
        ],
        out_specs=[
            pl.BlockSpec((_NB, D), lambda i: (i, 0)),
            pl.BlockSpec((_NB, 1), lambda i: (i, 0)),
            pl.BlockSpec((_NB, 1), lambda i: (i, 0)),
        ],
        out_shape=[
            jax.ShapeDtypeStruct((N, D), jnp.float32),
            jax.ShapeDtypeStruct((N, 1), jnp.float32),
            jax.ShapeDtypeStruct((N, 1), jnp.float32),
        ],
    )(x, w, a_s, a_d)


def _ve(we_all, ae_all):
    return (we_all * ae_all[:, None, :]).sum(-1)


def _eat_body(eat_ref, we_ref, ae_ref, out_ref):
    ve = _ve(we_ref[...], ae_ref[...])
    out_ref[...] = lax.dot_general(ve, eat_ref[...], (((1,), (0,)), ((), ())),
                                   preferred_element_type=jnp.float32)


def _k_eat(ea_t, we_all, ae_all):
    return pl.pallas_call(
        _eat_body,
        grid=(E // _EB,),
        in_specs=[
            pl.BlockSpec((DE, _EB), lambda i: (0, i)),
            pl.BlockSpec((3, DE, D), lambda i: (0, 0, 0)),
            pl.BlockSpec((3, D), lambda i: (0, 0)),
        ],
        out_specs=pl.BlockSpec((3, _EB), lambda i: (0, i)),
        out_shape=jax.ShapeDtypeStruct((3, E), jnp.float32),
    )(ea_t, we_all, ae_all)


def _lat_body(a_ref, lat_ref):
    deg = jnp.maximum(a_ref[0, 3] + a_ref[1, 3], 1.0)
    lat_ref[...] = jnp.concatenate(
        [(a_ref[0, l] + a_ref[1, l]) / deg for l in range(3)], axis=1)


def _k_lat(a_parts):
    return pl.pallas_call(
        _lat_body,
        grid=(N // _NB,),
        in_specs=[pl.BlockSpec((NC, 4, _NB, 1), lambda i: (0, 0, i, 0))],
        out_specs=pl.BlockSpec((_NB, 3), lambda i: (i, 0)),
        out_shape=jax.ShapeDtypeStruct((N, 3), jnp.float32),
    )(a_parts)


def _node_out(s_ref, d_ref, hw_ref, hs_ref, hd_ref, lat_ref, b_ref, layer):
    exs = hs_ref[...] + hd_ref[...] + lat_ref[:, layer:layer + 1]
    exs = jnp.exp(jnp.where(exs > 0, exs, 0.2 * exs))
    den = d_ref[0] + d_ref[1] + exs
    num = s_ref[0] + s_ref[1] + exs * hw_ref[...]
    return num / (den + 1e-16) + b_ref[...]


def _combine_body(s_ref, d_ref, hw_ref, hs_ref, hd_ref, lat_ref, b_ref, w_ref,
                  as_ref, ad_ref, hw2_ref, hs2_ref, hd2_ref, *, layer):
    h = jnp.maximum(_node_out(s_ref, d_ref, hw_ref, hs_ref, hd_ref, lat_ref,
                              b_ref, layer), 0.0)
    hw2 = jnp.dot(h, w_ref[...], preferred_element_type=jnp.float32)
    hw2_ref[...] = hw2
    hs2_ref[...] = jnp.dot(hw2, as_ref[...], preferred_element_type=jnp.float32)
    hd2_ref[...] = jnp.dot(hw2, ad_ref[...], preferred_element_type=jnp.float32)


def _k_combine(layer, s_parts, d_parts, hw, hs, hd, lat, b, w_next, as_next,
               ad_next):
    return pl.pallas_call(
        functools.partial(_combine_body, layer=layer),
        grid=(N // _NB,),
        in_specs=[
            pl.BlockSpec((NC, _NB, D), lambda i: (0, i, 0)),
            pl.BlockSpec((NC, _NB, 1), lambda i: (0, i, 0)),
            pl.BlockSpec((_NB, D), lambda i: (i, 0)),
            pl.BlockSpec((_NB, 1), lambda i: (i, 0)),
            pl.BlockSpec((_NB, 1), lambda i: (i, 0)),
            pl.BlockSpec((_NB, 3), lambda i: (i, 0)),
            pl.BlockSpec((1, D), lambda i: (0, 0)),
            pl.BlockSpec((D, D), lambda i: (0, 0)),
            pl.BlockSpec((D, 1), lambda i: (0, 0)),
            pl.BlockSpec((D, 1), lambda i: (0, 0)),
        ],
        out_specs=[
            pl.BlockSpec((_NB, D), lambda i: (i, 0)),
            pl.BlockSpec((_NB, 1), lambda i: (i, 0)),
            pl.BlockSpec((_NB, 1), lambda i: (i, 0)),
        ],
        out_shape=[
            jax.ShapeDtypeStruct((N, D), jnp.float32),
            jax.ShapeDtypeStruct((N, 1), jnp.float32),
            jax.ShapeDtypeStruct((N, 1), jnp.float32),
        ],
    )(s_parts, d_parts, hw, hs, hd, lat, b, w_next, as_next, ad_next)


def _final_body(s_ref, d_ref, hw_ref, hs_ref, hd_ref, lat_ref, b_ref, wl_ref,
                bl_ref, out_ref):
    h = _node_out(s_ref, d_ref, hw_ref, hs_ref, hd_ref, lat_ref, b_ref, 2)
    pooled = h.reshape(_FB // NPG, NPG, D).sum(axis=1)
    out_ref[...] = jnp.maximum(
        jnp.dot(pooled, wl_ref[...], preferred_element_type=jnp.float32)
        + bl_ref[...], 0.0)


def _k_final(s_parts, d_parts, hw, hs, hd, lat, b, wl, bl):
    return pl.pallas_call(
        _final_body,
        grid=(N // _FB,),
        in_specs=[
            pl.BlockSpec((NC, _FB, D), lambda i: (0, i, 0)),
            pl.BlockSpec((NC, _FB, 1), lambda i: (0, i, 0)),
            pl.BlockSpec((_FB, D), lambda i: (i, 0)),
            pl.BlockSpec((_FB, 1), lambda i: (i, 0)),
            pl.BlockSpec((_FB, 1), lambda i: (i, 0)),
            pl.BlockSpec((_FB, 3), lambda i: (i, 0)),
            pl.BlockSpec((1, D), lambda i: (0, 0)),
            pl.BlockSpec((D, 1), lambda i: (0, 0)),
            pl.BlockSpec((1, 1), lambda i: (0, 0)),
        ],
        out_specs=pl.BlockSpec((_FB // NPG, 1), lambda i: (i, 0)),
        out_shape=jax.ShapeDtypeStruct((NG, 1), jnp.float32),
    )(s_parts, d_parts, hw, hs, hd, lat, b, wl, bl)


def kernel(x, edge_index, edge_attr, W0, as0, ad0, We0, ae0, b0,
           W1, as1, ad1, We1, ae1, b1, W2, as2, ad2, We2, ae2, b2, Wl, bl):
    src = edge_index[0]
    dst = edge_index[1]
    we_all = jnp.stack([We0, We1, We2])
    ae_all = jnp.stack([ae0, ae1, ae2])

    eat = _k_eat(edge_attr.T, we_all, ae_all)
    a_parts = _attr_scatter(eat[0], eat[1], eat[2], dst)
    lat = _k_lat(a_parts.reshape(NC, 4, N, 1))

    hw, hs, hd = _k_hw(x, W0, as0.reshape(D, 1), ad0.reshape(D, 1))

    ws = [W1, W2]
    ass = [as1, as2]
    ads = [ad1, ad2]
    bs = [b0, b1, b2]
    for l in range(2):
        s_parts, d_parts = _edge_pass(hw, hs.reshape(N), hd.reshape(N),
                                      eat[l], src, dst)
        hw, hs, hd = _k_combine(l, s_parts, d_parts.reshape(NC, N, 1), hw,
                                hs, hd, lat, bs[l].reshape(1, D), ws[l],
                                ass[l].reshape(D, 1), ads[l].reshape(D, 1))
    s_parts, d_parts = _edge_pass(hw, hs.reshape(N), hd.reshape(N), eat[2],
                                  src, dst)
    return _k_final(s_parts, d_parts.reshape(NC, N, 1), hw, hs, hd, lat,
                    b2.reshape(1, D), Wl, bl.reshape(1, 1))


# speedup vs baseline: 1.6767x; 1.0536x over previous
import functools

import jax
import jax.numpy as jnp
from jax import lax
from jax.experimental import pallas as pl
from jax.experimental.pallas import tpu as pltpu
from jax.experimental.pallas import tpu_sc as plsc

N = 10080
E = 322560
D = 128
DE = 16
DEXT = 144
NPG = 84
NG = 120

NC = 2
NS = 16
NW = NC * NS
EPW = E // NW
B = 112
NCH = EPW // B
EB = 80
ENCH = EPW // EB
RPS = N // NS

_mesh = plsc.VectorSubcoreMesh(core_axis_name="c", subcore_axis_name="s")


def _attr_body(e0_hbm, e1_hbm, e2_hbm, dst_hbm, out_hbm,
               acc0, acc1, acc2, accd,
               dst0, dst1, f00, f01, f10, f11, f20, f21, ones_v, zb_v,
               isem0, isem1):
    c = lax.axis_index("c")
    s = lax.axis_index("s")
    wid = s * NC + c

    dst = (dst0, dst1)
    fs = ((f00, f01), (f10, f11), (f20, f21))
    e_hbm = (e0_hbm, e1_hbm, e2_hbm)
    accs = (acc0, acc1, acc2, accd)
    isem = (isem0, isem1)

    zeros16 = jnp.zeros((16,), jnp.float32)
    ones16 = jnp.full((16,), 1.0, jnp.float32)

    for g in range(B // 16):
        ones_v[pl.ds(g * 16, 16)] = ones16
    for g in range(6):
        zb_v[pl.ds(g * 16, 16)] = zeros16
    for k in range(7):
        idx = s + 16 * k

        @pl.when(idx < N // 96)
        def _():
            off = pl.multiple_of(idx * 96, 8)
            for a in accs:
                pltpu.sync_copy(zb_v, a.at[pl.ds(off, 96)])
    plsc.subcore_barrier()

    def _issue_idx(p, ch):
        base = wid * EPW + ch * B
        pltpu.async_copy(dst_hbm.at[pl.ds(base, B)], dst[p], isem[p])
        for l in range(3):
            pltpu.async_copy(e_hbm[l].at[pl.ds(base, B)], fs[l][p], isem[p])

    def _wait_idx(p, ch):
        base = wid * EPW + ch * B
        pltpu.make_async_copy(dst_hbm.at[pl.ds(base, B)], dst[p],
                              isem[p]).wait()
        for l in range(3):
            pltpu.make_async_copy(e_hbm[l].at[pl.ds(base, B)], fs[l][p],
                                  isem[p]).wait()

    _issue_idx(0, 0)
    _issue_idx(1, 1)

    def _iter(i, _):
        for p in (0, 1):
            ch = 2 * i + p
            _wait_idx(p, ch)
            for l in range(3):
                pltpu.sync_copy(fs[l][p], accs[l].at[dst[p]], add=True)
            pltpu.sync_copy(ones_v, accd.at[dst[p]], add=True)

            @pl.when(ch + 2 < NCH)
            def _():
                _issue_idx(p, ch + 2)
        return 0
    lax.fori_loop(0, NCH // 2, _iter, 0)

    plsc.subcore_barrier()

    @pl.when(s == 0)
    def _():
        for l in range(4):
            pltpu.sync_copy(accs[l], out_hbm.at[c, l])


_attr_scatter = functools.partial(
    pl.kernel,
    out_type=jax.ShapeDtypeStruct((NC, 4, N), jnp.float32),
    mesh=_mesh,
    scratch_types=(
        [pltpu.VMEM_SHARED((N,), jnp.float32)] * 4
        + [pltpu.VMEM((B,), jnp.int32)] * 2
        + [pltpu.VMEM((B,), jnp.float32)] * 7
        + [pltpu.VMEM((96,), jnp.float32)]
        + [pltpu.SemaphoreType.DMA] * 2
    ),
    compiler_params=pltpu.CompilerParams(use_tc_tiling_on_sc=False),
)(_attr_body)


def _edge_body(hw_hbm, hs_hbm, hd_hbm, eat_hbm, src_hbm, dst_hbm,
               out_hbm, outd_hbm,
               acc_sh, accd_sh,
               src0, src1, dst0, dst1, sdst0, sdst1, eat0, eat1,
               hsg0, hsg1, hdg0, hdg1, ex0, ex1, rows0, rows1, sca0, sca1,
               isem0, isem1, gsem0, gsem1, hsem0, hsem1, ssem0, ssem1):
    c = lax.axis_index("c")
    s = lax.axis_index("s")
    wid = s * NC + c

    src = (src0, src1)
    dst = (dst0, dst1)
    sdst = (sdst0, sdst1)
    eat = (eat0, eat1)
    hsg = (hsg0, hsg1)
    hdg = (hdg0, hdg1)
    ex = (ex0, ex1)
    rows = (rows0, rows1)
    sca = (sca0, sca1)
    isem = (isem0, isem1)
    gsem = (gsem0, gsem1)
    hsem = (hsem0, hsem1)
    ssem = (ssem0, ssem1)

    zeros16 = jnp.zeros((16,), jnp.float32)

    def _zfill(i, _):
        for j in range(8):
            sca0[i, pl.ds(j * 16, 16)] = zeros16
        return 0
    lax.fori_loop(0, EB, _zfill, 0)
    for g in range(6):
        ex0[pl.ds(g * 16, 16)] = zeros16
    for k in range(7):
        pltpu.sync_copy(sca0, acc_sh.at[pl.ds(s * RPS + k * EB, EB)])
    pltpu.sync_copy(sca0.at[pl.ds(0, RPS - 7 * EB)],
                    acc_sh.at[pl.ds(s * RPS + 7 * EB, RPS - 7 * EB)])
    for k in range(7):
        idx = s + 16 * k

        @pl.when(idx < N // 96)
        def _():
            pltpu.sync_copy(ex0.at[pl.ds(0, 96)],
                            accd_sh.at[pl.ds(pl.multiple_of(idx * 96, 8), 96)])
    plsc.subcore_barrier()

    def _issue_idx(p, ch):
        base = wid * EPW + ch * EB
        pltpu.async_copy(src_hbm.at[pl.ds(base, EB)], src[p], isem[p])
        pltpu.async_copy(dst_hbm.at[pl.ds(base, EB)], dst[p], isem[p])
        pltpu.async_copy(eat_hbm.at[pl.ds(base, EB)], eat[p], isem[p])

    def _wait_idx(p, ch):
        base = wid * EPW + ch * EB
        pltpu.make_async_copy(src_hbm.at[pl.ds(base, EB)], src[p],
                              isem[p]).wait()
        pltpu.make_async_copy(dst_hbm.at[pl.ds(base, EB)], dst[p],
                              isem[p]).wait()
        pltpu.make_async_copy(eat_hbm.at[pl.ds(base, EB)], eat[p],
                              isem[p]).wait()

    def _issue_gathers(p):
        pltpu.async_copy(hw_hbm.at[src[p]], rows[p], gsem[p])
        pltpu.async_copy(hs_hbm.at[src[p]], hsg[p], hsem[p])
        pltpu.async_copy(hd_hbm.at[dst[p]], hdg[p], hsem[p])

    def _wait_scatters(p):
        pltpu.make_async_copy(sca[p], acc_sh.at[sdst[p]], ssem[p]).wait()
        pltpu.make_async_copy(ex[p].at[pl.ds(0, EB)], accd_sh.at[sdst[p]],
                              ssem[p]).wait()

    _issue_idx(0, 0)
    _wait_idx(0, 0)
    _issue_gathers(0)
    _issue_idx(1, 1)

    def _iter(i, _):
        for p in (0, 1):
            q = 1 - p
            ch = 2 * i + p

            @pl.when(ch + 1 < ENCH)
            def _():
                _wait_idx(q, ch + 1)
                _issue_gathers(q)

            @pl.when(ch >= 2)
            def _():
                _wait_scatters(p)

            pltpu.make_async_copy(hs_hbm.at[src[p]], hsg[p], hsem[p]).wait()
            pltpu.make_async_copy(hd_hbm.at[dst[p]], hdg[p], hsem[p]).wait()
            for g in range(EB // 16):
                sl = pl.ds(g * 16, 16)
                a = hsg[p][sl] + hdg[p][sl] + eat[p][sl]
                a = jnp.where(a > 0, a, 0.2 * a)
                ex[p][sl] = jnp.exp(a)
            pltpu.make_async_copy(hw_hbm.at[src[p]], rows[p], gsem[p]).wait()
            for g in range(EB // 16):
                sl = pl.ds(g * 16, 16)
                sdst[p][sl] = dst[p][sl]

            def _edge(b2, _):
                b = 2 * b2
                exv = ex[p][pl.ds(b, 16)]
                exb = exv[0]
                exb1 = exv[1]
                for j in range(8):
                    js = pl.ds(j * 16, 16)
                    sca[p][b, js] = rows[p][b, js] * exb
                for j in range(8):
                    js = pl.ds(j * 16, 16)
                    sca[p][b + 1, js] = rows[p][b + 1, js] * exb1
                return 0
            lax.fori_loop(0, EB // 2, _edge, 0)

            pltpu.async_copy(sca[p], acc_sh.at[sdst[p]], ssem[p], add=True)
            pltpu.async_copy(ex[p].at[pl.ds(0, EB)], accd_sh.at[sdst[p]],
                             ssem[p], add=True)

            @pl.when(ch + 2 < ENCH)
            def _():
                _issue_idx(p, ch + 2)
        return 0
    lax.fori_loop(0, ENCH // 2, _iter, 0)

    for p in (0, 1):
        _wait_scatters(p)
    plsc.subcore_barrier()

    @pl.when(s == 0)
    def _():
        pltpu.sync_copy(acc_sh, out_hbm.at[c])
        pltpu.sync_copy(accd_sh, outd_hbm.at[c])


_edge_pass = functools.partial(
    pl.kernel,
    out_type=(jax.ShapeDtypeStruct((NC, N, D), jnp.float32),
              jax.ShapeDtypeStruct((NC, N), jnp.float32)),
    mesh=_mesh,
    scratch_types=(
        [pltpu.VMEM_SHARED((N, D), jnp.float32),
         pltpu.VMEM_SHARED((N,), jnp.float32)]
        + [pltpu.VMEM((EB,), jnp.int32)] * 6
        + [pltpu.VMEM((EB,), jnp.float32)] * 6
        + [pltpu.VMEM((EB + 16,), jnp.float32)] * 2
        + [pltpu.VMEM((EB, D), jnp.float32)] * 4
        + [pltpu.SemaphoreType.DMA] * 8
    ),
    compiler_params=pltpu.CompilerParams(use_tc_tiling_on_sc=False),
)(_edge_body)


_NB = 1008
_EB = 5120
_FB = 2016


def _hw_body(x_ref, w_ref, as_ref, ad_ref, hw_ref, hs_ref, hd_ref):
    hw = jnp.dot(x_ref[...], w_ref[...], preferred_element_type=jnp.float32)
    hw_ref[...] = hw
    hs_ref[...] = jnp.dot(hw, as_ref[...], preferred_element_type=jnp.float32)
    hd_ref[...] = jnp.dot(hw, ad_ref[...], preferred_element_type=jnp.float32)


def _k_hw(x, w, a_s, a_d):
    return pl.pallas_call(
        _hw_body,
        grid=(N // _NB,),
        in_specs=[
            pl.BlockSpec((_NB, D), lambda i: (i, 0)),
            pl.BlockSpec((D, D), lambda i: (0, 0)),
            pl.BlockSpec((D, 1), lambda i: (0, 0)),
            pl.BlockSpec((D, 1), lambda i: (0, 0)),
        ],
        out_specs=[
            pl.BlockSpec((_NB, D), lambda i: (i, 0)),
            pl.BlockSpec((_NB, 1), lambda i: (i, 0)),
            pl.BlockSpec((_NB, 1), lambda i: (i, 0)),
        ],
        out_shape=[
            jax.ShapeDtypeStruct((N, D), jnp.float32),
            jax.ShapeDtypeStruct((N, 1), jnp.float32),
            jax.ShapeDtypeStruct((N, 1), jnp.float32),
        ],
    )(x, w, a_s, a_d)


def _ve(we_all, ae_all):
    return (we_all * ae_all[:, None, :]).sum(-1)


def _eat_body(eat_ref, we_ref, ae_ref, out_ref):
    ve = _ve(we_ref[...], ae_ref[...])
    out_ref[...] = lax.dot_general(ve, eat_ref[...], (((1,), (0,)), ((), ())),
                                   preferred_element_type=jnp.float32)


def _k_eat(ea_t, we_all, ae_all):
    return pl.pallas_call(
        _eat_body,
        grid=(E // _EB,),
        in_specs=[
            pl.BlockSpec((DE, _EB), lambda i: (0, i)),
            pl.BlockSpec((3, DE, D), lambda i: (0, 0, 0)),
            pl.BlockSpec((3, D), lambda i: (0, 0)),
        ],
        out_specs=pl.BlockSpec((3, _EB), lambda i: (0, i)),
        out_shape=jax.ShapeDtypeStruct((3, E), jnp.float32),
    )(ea_t, we_all, ae_all)


def _lat_body(a_ref, lat_ref):
    deg = jnp.maximum(a_ref[0, 3] + a_ref[1, 3], 1.0)
    lat_ref[...] = jnp.concatenate(
        [(a_ref[0, l] + a_ref[1, l]) / deg for l in range(3)], axis=1)


def _k_lat(a_parts):
    return pl.pallas_call(
        _lat_body,
        grid=(N // _NB,),
        in_specs=[pl.BlockSpec((NC, 4, _NB, 1), lambda i: (0, 0, i, 0))],
        out_specs=pl.BlockSpec((_NB, 3), lambda i: (i, 0)),
        out_shape=jax.ShapeDtypeStruct((N, 3), jnp.float32),
    )(a_parts)


def _node_out(s_ref, d_ref, hw_ref, hs_ref, hd_ref, lat_ref, b_ref, layer):
    exs = hs_ref[...] + hd_ref[...] + lat_ref[:, layer:layer + 1]
    exs = jnp.exp(jnp.where(exs > 0, exs, 0.2 * exs))
    den = d_ref[0] + d_ref[1] + exs
    num = s_ref[0] + s_ref[1] + exs * hw_ref[...]
    return num / (den + 1e-16) + b_ref[...]


def _combine_body(s_ref, d_ref, hw_ref, hs_ref, hd_ref, lat_ref, b_ref, w_ref,
                  as_ref, ad_ref, hw2_ref, hs2_ref, hd2_ref, *, layer):
    h = jnp.maximum(_node_out(s_ref, d_ref, hw_ref, hs_ref, hd_ref, lat_ref,
                              b_ref, layer), 0.0)
    hw2 = jnp.dot(h, w_ref[...], preferred_element_type=jnp.float32)
    hw2_ref[...] = hw2
    hs2_ref[...] = jnp.dot(hw2, as_ref[...], preferred_element_type=jnp.float32)
    hd2_ref[...] = jnp.dot(hw2, ad_ref[...], preferred_element_type=jnp.float32)


def _k_combine(layer, s_parts, d_parts, hw, hs, hd, lat, b, w_next, as_next,
               ad_next):
    return pl.pallas_call(
        functools.partial(_combine_body, layer=layer),
        grid=(N // _NB,),
        in_specs=[
            pl.BlockSpec((NC, _NB, D), lambda i: (0, i, 0)),
            pl.BlockSpec((NC, _NB, 1), lambda i: (0, i, 0)),
            pl.BlockSpec((_NB, D), lambda i: (i, 0)),
            pl.BlockSpec((_NB, 1), lambda i: (i, 0)),
            pl.BlockSpec((_NB, 1), lambda i: (i, 0)),
            pl.BlockSpec((_NB, 3), lambda i: (i, 0)),
            pl.BlockSpec((1, D), lambda i: (0, 0)),
            pl.BlockSpec((D, D), lambda i: (0, 0)),
            pl.BlockSpec((D, 1), lambda i: (0, 0)),
            pl.BlockSpec((D, 1), lambda i: (0, 0)),
        ],
        out_specs=[
            pl.BlockSpec((_NB, D), lambda i: (i, 0)),
            pl.BlockSpec((_NB, 1), lambda i: (i, 0)),
            pl.BlockSpec((_NB, 1), lambda i: (i, 0)),
        ],
        out_shape=[
            jax.ShapeDtypeStruct((N, D), jnp.float32),
            jax.ShapeDtypeStruct((N, 1), jnp.float32),
            jax.ShapeDtypeStruct((N, 1), jnp.float32),
        ],
    )(s_parts, d_parts, hw, hs, hd, lat, b, w_next, as_next, ad_next)


def _final_body(s_ref, d_ref, hw_ref, hs_ref, hd_ref, lat_ref, b_ref, wl_ref,
                bl_ref, out_ref):
    h = _node_out(s_ref, d_ref, hw_ref, hs_ref, hd_ref, lat_ref, b_ref, 2)
    pooled = h.reshape(_FB // NPG, NPG, D).sum(axis=1)
    out_ref[...] = jnp.maximum(
        jnp.dot(pooled, wl_ref[...], preferred_element_type=jnp.float32)
        + bl_ref[...], 0.0)


def _k_final(s_parts, d_parts, hw, hs, hd, lat, b, wl, bl):
    return pl.pallas_call(
        _final_body,
        grid=(N // _FB,),
        in_specs=[
            pl.BlockSpec((NC, _FB, D), lambda i: (0, i, 0)),
            pl.BlockSpec((NC, _FB, 1), lambda i: (0, i, 0)),
            pl.BlockSpec((_FB, D), lambda i: (i, 0)),
            pl.BlockSpec((_FB, 1), lambda i: (i, 0)),
            pl.BlockSpec((_FB, 1), lambda i: (i, 0)),
            pl.BlockSpec((_FB, 3), lambda i: (i, 0)),
            pl.BlockSpec((1, D), lambda i: (0, 0)),
            pl.BlockSpec((D, 1), lambda i: (0, 0)),
            pl.BlockSpec((1, 1), lambda i: (0, 0)),
        ],
        out_specs=pl.BlockSpec((_FB // NPG, 1), lambda i: (i, 0)),
        out_shape=jax.ShapeDtypeStruct((NG, 1), jnp.float32),
    )(s_parts, d_parts, hw, hs, hd, lat, b, wl, bl)


def kernel(x, edge_index, edge_attr, W0, as0, ad0, We0, ae0, b0,
           W1, as1, ad1, We1, ae1, b1, W2, as2, ad2, We2, ae2, b2, Wl, bl):
    src = edge_index[0]
    dst = edge_index[1]
    we_all = jnp.stack([We0, We1, We2])
    ae_all = jnp.stack([ae0, ae1, ae2])

    eat = _k_eat(edge_attr.T, we_all, ae_all)
    a_parts = _attr_scatter(eat[0], eat[1], eat[2], dst)
    lat = _k_lat(a_parts.reshape(NC, 4, N, 1))

    hw, hs, hd = _k_hw(x, W0, as0.reshape(D, 1), ad0.reshape(D, 1))

    ws = [W1, W2]
    ass = [as1, as2]
    ads = [ad1, ad2]
    bs = [b0, b1, b2]
    for l in range(2):
        s_parts, d_parts = _edge_pass(hw, hs.reshape(N), hd.reshape(N),
                                      eat[l], src, dst)
        hw, hs, hd = _k_combine(l, s_parts, d_parts.reshape(NC, N, 1), hw,
                                hs, hd, lat, bs[l].reshape(1, D), ws[l],
                                ass[l].reshape(D, 1), ads[l].reshape(D, 1))
    s_parts, d_parts = _edge_pass(hw, hs.reshape(N), hd.reshape(N), eat[2],
                                  src, dst)
    return _k_final(s_parts, d_parts.reshape(NC, N, 1), hw, hs, hd, lat,
                    b2.reshape(1, D), Wl, bl.reshape(1, 1))


# speedup vs baseline: 1.7167x; 1.0238x over previous
import functools

import jax
import jax.numpy as jnp
from jax import lax
from jax.experimental import pallas as pl
from jax.experimental.pallas import tpu as pltpu
from jax.experimental.pallas import tpu_sc as plsc

N = 10080
E = 322560
D = 128
DE = 16
DEXT = 144
NPG = 84
NG = 120

NC = 2
NS = 16
NW = NC * NS
EPW = E // NW
B = 112
NCH = EPW // B
EB = 80
ENCH = EPW // EB
RPS = N // NS

_mesh = plsc.VectorSubcoreMesh(core_axis_name="c", subcore_axis_name="s")


def _attr_body(e0_hbm, e1_hbm, e2_hbm, dst_hbm, out_hbm,
               acc0, acc1, acc2, accd,
               dst0, dst1, f00, f01, f10, f11, f20, f21, ones_v, zb_v,
               isem0, isem1):
    c = lax.axis_index("c")
    s = lax.axis_index("s")
    wid = s * NC + c

    dst = (dst0, dst1)
    fs = ((f00, f01), (f10, f11), (f20, f21))
    e_hbm = (e0_hbm, e1_hbm, e2_hbm)
    accs = (acc0, acc1, acc2, accd)
    isem = (isem0, isem1)

    zeros16 = jnp.zeros((16,), jnp.float32)
    ones16 = jnp.full((16,), 1.0, jnp.float32)

    for g in range(B // 16):
        ones_v[pl.ds(g * 16, 16)] = ones16
    for g in range(6):
        zb_v[pl.ds(g * 16, 16)] = zeros16
    for k in range(7):
        idx = s + 16 * k

        @pl.when(idx < N // 96)
        def _():
            off = pl.multiple_of(idx * 96, 8)
            for a in accs:
                pltpu.sync_copy(zb_v, a.at[pl.ds(off, 96)])
    plsc.subcore_barrier()

    def _issue_idx(p, ch):
        base = wid * EPW + ch * B
        pltpu.async_copy(dst_hbm.at[pl.ds(base, B)], dst[p], isem[p])
        for l in range(3):
            pltpu.async_copy(e_hbm[l].at[pl.ds(base, B)], fs[l][p], isem[p])

    def _wait_idx(p, ch):
        base = wid * EPW + ch * B
        pltpu.make_async_copy(dst_hbm.at[pl.ds(base, B)], dst[p],
                              isem[p]).wait()
        for l in range(3):
            pltpu.make_async_copy(e_hbm[l].at[pl.ds(base, B)], fs[l][p],
                                  isem[p]).wait()

    _issue_idx(0, 0)
    _issue_idx(1, 1)

    def _iter(i, _):
        for p in (0, 1):
            ch = 2 * i + p
            _wait_idx(p, ch)
            for l in range(3):
                pltpu.sync_copy(fs[l][p], accs[l].at[dst[p]], add=True)
            pltpu.sync_copy(ones_v, accd.at[dst[p]], add=True)

            @pl.when(ch + 2 < NCH)
            def _():
                _issue_idx(p, ch + 2)
        return 0
    lax.fori_loop(0, NCH // 2, _iter, 0)

    plsc.subcore_barrier()

    @pl.when(s == 0)
    def _():
        for l in range(4):
            pltpu.sync_copy(accs[l], out_hbm.at[c, l])


_attr_scatter = functools.partial(
    pl.kernel,
    out_type=jax.ShapeDtypeStruct((NC, 4, N), jnp.float32),
    mesh=_mesh,
    scratch_types=(
        [pltpu.VMEM_SHARED((N,), jnp.float32)] * 4
        + [pltpu.VMEM((B,), jnp.int32)] * 2
        + [pltpu.VMEM((B,), jnp.float32)] * 7
        + [pltpu.VMEM((96,), jnp.float32)]
        + [pltpu.SemaphoreType.DMA] * 2
    ),
    compiler_params=pltpu.CompilerParams(use_tc_tiling_on_sc=False),
)(_attr_body)


def _edge_body(hw_hbm, hs_hbm, hd_hbm, eat_hbm, src_hbm, dst_hbm,
               out_hbm, outd_hbm,
               acc_sh, accd_sh,
               src0, src1, dst0, dst1, sdst0, sdst1, eat0, eat1,
               hsg0, hsg1, hdg0, hdg1, ex0, ex1, rows0, rows1, sca0, sca1,
               isem0, isem1, gsem0, gsem1, hsem0, hsem1, ssem0, ssem1):
    c = lax.axis_index("c")
    s = lax.axis_index("s")
    wid = s * NC + c

    src = (src0, src1)
    dst = (dst0, dst1)
    sdst = (sdst0, sdst1)
    eat = (eat0, eat1)
    hsg = (hsg0, hsg1)
    hdg = (hdg0, hdg1)
    ex = (ex0, ex1)
    rows = (rows0, rows1)
    sca = (sca0, sca1)
    isem = (isem0, isem1)
    gsem = (gsem0, gsem1)
    hsem = (hsem0, hsem1)
    ssem = (ssem0, ssem1)

    zeros16 = jnp.zeros((16,), jnp.float32)

    def _zfill(i, _):
        for j in range(8):
            sca0[i, pl.ds(j * 16, 16)] = zeros16
        return 0
    lax.fori_loop(0, EB, _zfill, 0)
    for g in range(6):
        ex0[pl.ds(g * 16, 16)] = zeros16
    for k in range(7):
        pltpu.sync_copy(sca0, acc_sh.at[pl.ds(s * RPS + k * EB, EB)])
    pltpu.sync_copy(sca0.at[pl.ds(0, RPS - 7 * EB)],
                    acc_sh.at[pl.ds(s * RPS + 7 * EB, RPS - 7 * EB)])
    for k in range(7):
        idx = s + 16 * k

        @pl.when(idx < N // 96)
        def _():
            pltpu.sync_copy(ex0.at[pl.ds(0, 96)],
                            accd_sh.at[pl.ds(pl.multiple_of(idx * 96, 8), 96)])
    plsc.subcore_barrier()

    def _issue_idx(p, ch):
        base = wid * EPW + ch * EB
        pltpu.async_copy(src_hbm.at[pl.ds(base, EB)], src[p], isem[p])
        pltpu.async_copy(dst_hbm.at[pl.ds(base, EB)], dst[p], isem[p])
        pltpu.async_copy(eat_hbm.at[pl.ds(base, EB)], eat[p], isem[p])

    def _wait_idx(p, ch):
        base = wid * EPW + ch * EB
        pltpu.make_async_copy(src_hbm.at[pl.ds(base, EB)], src[p],
                              isem[p]).wait()
        pltpu.make_async_copy(dst_hbm.at[pl.ds(base, EB)], dst[p],
                              isem[p]).wait()
        pltpu.make_async_copy(eat_hbm.at[pl.ds(base, EB)], eat[p],
                              isem[p]).wait()

    def _issue_gathers(p):
        pltpu.async_copy(hw_hbm.at[src[p]], rows[p], gsem[p])
        pltpu.async_copy(hs_hbm.at[src[p]], hsg[p], hsem[p])
        pltpu.async_copy(hd_hbm.at[dst[p]], hdg[p], hsem[p])

    def _wait_scatters(p):
        pltpu.make_async_copy(sca[p], acc_sh.at[sdst[p]], ssem[p]).wait()
        pltpu.make_async_copy(ex[p].at[pl.ds(0, EB)], accd_sh.at[sdst[p]],
                              ssem[p]).wait()

    _issue_idx(0, 0)
    _wait_idx(0, 0)
    _issue_gathers(0)
    _issue_idx(1, 1)

    def _iter(i, _):
        for p in (0, 1):
            q = 1 - p
            ch = 2 * i + p

            @pl.when(ch + 1 < ENCH)
            def _():
                _wait_idx(q, ch + 1)
                _issue_gathers(q)

            @pl.when(ch >= 2)
            def _():
                _wait_scatters(p)

            pltpu.make_async_copy(hs_hbm.at[src[p]], hsg[p], hsem[p]).wait()
            pltpu.make_async_copy(hd_hbm.at[dst[p]], hdg[p], hsem[p]).wait()
            for g in range(EB // 16):
                sl = pl.ds(g * 16, 16)
                a = hsg[p][sl] + hdg[p][sl] + eat[p][sl]
                a = jnp.where(a > 0, a, 0.2 * a)
                ex[p][sl] = jnp.exp(a)
            pltpu.make_async_copy(hw_hbm.at[src[p]], rows[p], gsem[p]).wait()
            for g in range(EB // 16):
                sl = pl.ds(g * 16, 16)
                sdst[p][sl] = dst[p][sl]

            def _edge(b4, _):
                b = 4 * b4
                exv = ex[p][pl.ds(b, 16)]
                for u in range(4):
                    exb = exv[u]
                    for j in range(8):
                        js = pl.ds(j * 16, 16)
                        sca[p][b + u, js] = rows[p][b + u, js] * exb
                return 0
            lax.fori_loop(0, EB // 4, _edge, 0)

            pltpu.async_copy(sca[p], acc_sh.at[sdst[p]], ssem[p], add=True)
            pltpu.async_copy(ex[p].at[pl.ds(0, EB)], accd_sh.at[sdst[p]],
                             ssem[p], add=True)

            @pl.when(ch + 2 < ENCH)
            def _():
                _issue_idx(p, ch + 2)
        return 0
    lax.fori_loop(0, ENCH // 2, _iter, 0)

    for p in (0, 1):
        _wait_scatters(p)
    plsc.subcore_barrier()

    @pl.when(s == 0)
    def _():
        pltpu.sync_copy(acc_sh, out_hbm.at[c])
        pltpu.sync_copy(accd_sh, outd_hbm.at[c])


_edge_pass = functools.partial(
    pl.kernel,
    out_type=(jax.ShapeDtypeStruct((NC, N, D), jnp.float32),
              jax.ShapeDtypeStruct((NC, N), jnp.float32)),
    mesh=_mesh,
    scratch_types=(
        [pltpu.VMEM_SHARED((N, D), jnp.float32),
         pltpu.VMEM_SHARED((N,), jnp.float32)]
        + [pltpu.VMEM((EB,), jnp.int32)] * 6
        + [pltpu.VMEM((EB,), jnp.float32)] * 6
        + [pltpu.VMEM((EB + 16,), jnp.float32)] * 2
        + [pltpu.VMEM((EB, D), jnp.float32)] * 4
        + [pltpu.SemaphoreType.DMA] * 8
    ),
    compiler_params=pltpu.CompilerParams(use_tc_tiling_on_sc=False),
)(_edge_body)


_NB = 1008
_EB = 5120
_FB = 2016


def _hw_body(x_ref, w_ref, as_ref, ad_ref, hw_ref, hs_ref, hd_ref):
    hw = jnp.dot(x_ref[...], w_ref[...], preferred_element_type=jnp.float32)
    hw_ref[...] = hw
    hs_ref[...] = jnp.dot(hw, as_ref[...], preferred_element_type=jnp.float32)
    hd_ref[...] = jnp.dot(hw, ad_ref[...], preferred_element_type=jnp.float32)


def _k_hw(x, w, a_s, a_d):
    return pl.pallas_call(
        _hw_body,
        grid=(N // _NB,),
        in_specs=[
            pl.BlockSpec((_NB, D), lambda i: (i, 0)),
            pl.BlockSpec((D, D), lambda i: (0, 0)),
            pl.BlockSpec((D, 1), lambda i: (0, 0)),
            pl.BlockSpec((D, 1), lambda i: (0, 0)),
        ],
        out_specs=[
            pl.BlockSpec((_NB, D), lambda i: (i, 0)),
            pl.BlockSpec((_NB, 1), lambda i: (i, 0)),
            pl.BlockSpec((_NB, 1), lambda i: (i, 0)),
        ],
        out_shape=[
            jax.ShapeDtypeStruct((N, D), jnp.float32),
            jax.ShapeDtypeStruct((N, 1), jnp.float32),
            jax.ShapeDtypeStruct((N, 1), jnp.float32),
        ],
    )(x, w, a_s, a_d)


def _ve(we_all, ae_all):
    return (we_all * ae_all[:, None, :]).sum(-1)


def _eat_body(eat_ref, we_ref, ae_ref, out_ref):
    ve = _ve(we_ref[...], ae_ref[...])
    out_ref[...] = lax.dot_general(ve, eat_ref[...], (((1,), (0,)), ((), ())),
                                   preferred_element_type=jnp.float32)


def _k_eat(ea_t, we_all, ae_all):
    return pl.pallas_call(
        _eat_body,
        grid=(E // _EB,),
        in_specs=[
            pl.BlockSpec((DE, _EB), lambda i: (0, i)),
            pl.BlockSpec((3, DE, D), lambda i: (0, 0, 0)),
            pl.BlockSpec((3, D), lambda i: (0, 0)),
        ],
        out_specs=pl.BlockSpec((3, _EB), lambda i: (0, i)),
        out_shape=jax.ShapeDtypeStruct((3, E), jnp.float32),
    )(ea_t, we_all, ae_all)


def _lat_body(a_ref, lat_ref):
    deg = jnp.maximum(a_ref[0, 3] + a_ref[1, 3], 1.0)
    lat_ref[...] = jnp.concatenate(
        [(a_ref[0, l] + a_ref[1, l]) / deg for l in range(3)], axis=1)


def _k_lat(a_parts):
    return pl.pallas_call(
        _lat_body,
        grid=(N // _NB,),
        in_specs=[pl.BlockSpec((NC, 4, _NB, 1), lambda i: (0, 0, i, 0))],
        out_specs=pl.BlockSpec((_NB, 3), lambda i: (i, 0)),
        out_shape=jax.ShapeDtypeStruct((N, 3), jnp.float32),
    )(a_parts)


def _node_out(s_ref, d_ref, hw_ref, hs_ref, hd_ref, lat_ref, b_ref, layer):
    exs = hs_ref[...] + hd_ref[...] + lat_ref[:, layer:layer + 1]
    exs = jnp.exp(jnp.where(exs > 0, exs, 0.2 * exs))
    den = d_ref[0] + d_ref[1] + exs
    num = s_ref[0] + s_ref[1] + exs * hw_ref[...]
    return num / (den + 1e-16) + b_ref[...]


def _combine_body(s_ref, d_ref, hw_ref, hs_ref, hd_ref, lat_ref, b_ref, w_ref,
                  as_ref, ad_ref, hw2_ref, hs2_ref, hd2_ref, *, layer):
    h = jnp.maximum(_node_out(s_ref, d_ref, hw_ref, hs_ref, hd_ref, lat_ref,
                              b_ref, layer), 0.0)
    hw2 = jnp.dot(h, w_ref[...], preferred_element_type=jnp.float32)
    hw2_ref[...] = hw2
    hs2_ref[...] = jnp.dot(hw2, as_ref[...], preferred_element_type=jnp.float32)
    hd2_ref[...] = jnp.dot(hw2, ad_ref[...], preferred_element_type=jnp.float32)


def _k_combine(layer, s_parts, d_parts, hw, hs, hd, lat, b, w_next, as_next,
               ad_next):
    return pl.pallas_call(
        functools.partial(_combine_body, layer=layer),
        grid=(N // _NB,),
        in_specs=[
            pl.BlockSpec((NC, _NB, D), lambda i: (0, i, 0)),
            pl.BlockSpec((NC, _NB, 1), lambda i: (0, i, 0)),
            pl.BlockSpec((_NB, D), lambda i: (i, 0)),
            pl.BlockSpec((_NB, 1), lambda i: (i, 0)),
            pl.BlockSpec((_NB, 1), lambda i: (i, 0)),
            pl.BlockSpec((_NB, 3), lambda i: (i, 0)),
            pl.BlockSpec((1, D), lambda i: (0, 0)),
            pl.BlockSpec((D, D), lambda i: (0, 0)),
            pl.BlockSpec((D, 1), lambda i: (0, 0)),
            pl.BlockSpec((D, 1), lambda i: (0, 0)),
        ],
        out_specs=[
            pl.BlockSpec((_NB, D), lambda i: (i, 0)),
            pl.BlockSpec((_NB, 1), lambda i: (i, 0)),
            pl.BlockSpec((_NB, 1), lambda i: (i, 0)),
        ],
        out_shape=[
            jax.ShapeDtypeStruct((N, D), jnp.float32),
            jax.ShapeDtypeStruct((N, 1), jnp.float32),
            jax.ShapeDtypeStruct((N, 1), jnp.float32),
        ],
    )(s_parts, d_parts, hw, hs, hd, lat, b, w_next, as_next, ad_next)


def _final_body(s_ref, d_ref, hw_ref, hs_ref, hd_ref, lat_ref, b_ref, wl_ref,
                bl_ref, out_ref):
    h = _node_out(s_ref, d_ref, hw_ref, hs_ref, hd_ref, lat_ref, b_ref, 2)
    pooled = h.reshape(_FB // NPG, NPG, D).sum(axis=1)
    out_ref[...] = jnp.maximum(
        jnp.dot(pooled, wl_ref[...], preferred_element_type=jnp.float32)
        + bl_ref[...], 0.0)


def _k_final(s_parts, d_parts, hw, hs, hd, lat, b, wl, bl):
    return pl.pallas_call(
        _final_body,
        grid=(N // _FB,),
        in_specs=[
            pl.BlockSpec((NC, _FB, D), lambda i: (0, i, 0)),
            pl.BlockSpec((NC, _FB, 1), lambda i: (0, i, 0)),
            pl.BlockSpec((_FB, D), lambda i: (i, 0)),
            pl.BlockSpec((_FB, 1), lambda i: (i, 0)),
            pl.BlockSpec((_FB, 1), lambda i: (i, 0)),
            pl.BlockSpec((_FB, 3), lambda i: (i, 0)),
            pl.BlockSpec((1, D), lambda i: (0, 0)),
            pl.BlockSpec((D, 1), lambda i: (0, 0)),
            pl.BlockSpec((1, 1), lambda i: (0, 0)),
        ],
        out_specs=pl.BlockSpec((_FB // NPG, 1), lambda i: (i, 0)),
        out_shape=jax.ShapeDtypeStruct((NG, 1), jnp.float32),
    )(s_parts, d_parts, hw, hs, hd, lat, b, wl, bl)


def kernel(x, edge_index, edge_attr, W0, as0, ad0, We0, ae0, b0,
           W1, as1, ad1, We1, ae1, b1, W2, as2, ad2, We2, ae2, b2, Wl, bl):
    src = edge_index[0]
    dst = edge_index[1]
    we_all = jnp.stack([We0, We1, We2])
    ae_all = jnp.stack([ae0, ae1, ae2])

    eat = _k_eat(edge_attr.T, we_all, ae_all)
    a_parts = _attr_scatter(eat[0], eat[1], eat[2], dst)
    lat = _k_lat(a_parts.reshape(NC, 4, N, 1))

    hw, hs, hd = _k_hw(x, W0, as0.reshape(D, 1), ad0.reshape(D, 1))

    ws = [W1, W2]
    ass = [as1, as2]
    ads = [ad1, ad2]
    bs = [b0, b1, b2]
    for l in range(2):
        s_parts, d_parts = _edge_pass(hw, hs.reshape(N), hd.reshape(N),
                                      eat[l], src, dst)
        hw, hs, hd = _k_combine(l, s_parts, d_parts.reshape(NC, N, 1), hw,
                                hs, hd, lat, bs[l].reshape(1, D), ws[l],
                                ass[l].reshape(D, 1), ads[l].reshape(D, 1))
    s_parts, d_parts = _edge_pass(hw, hs.reshape(N), hd.reshape(N), eat[2],
                                  src, dst)
    return _k_final(s_parts, d_parts.reshape(NC, N, 1), hw, hs, hd, lat,
                    b2.reshape(1, D), Wl, bl.reshape(1, 1))
